# baseline (speedup 1.0000x reference)
import functools

import jax
import jax.numpy as jnp
from jax import lax
from jax.experimental import pallas as pl
from jax.experimental.pallas import tpu as pltpu
from jax.experimental.pallas import tpu_sc as plsc

BATCH = 16384
F = 26
V = 100000
D = 16

NC, NS, L = 2, 16, 16
NW = NC * NS
NP = F * D
PPW = NP // NW
OC = 4096
NOC = BATCH // OC

_mesh = plsc.VectorSubcoreMesh(
    core_axis_name="c", subcore_axis_name="s", num_cores=NC, num_subcores=NS
)


@functools.partial(
    pl.kernel,
    out_type=jax.ShapeDtypeStruct((NP, BATCH), jnp.float32),
    mesh=_mesh,
    scratch_types=[
        pltpu.VMEM((V,), jnp.float32),
        pltpu.VMEM((BATCH,), jnp.int32),
        pltpu.VMEM((2, OC), jnp.float32),
        pltpu.SemaphoreType.DMA,
        pltpu.SemaphoreType.DMA((2,)),
    ],
    compiler_params=pltpu.CompilerParams(
        use_tc_tiling_on_sc=True, needs_layout_passes=False
    ),
)
def _emb(xt_hbm, tw_hbm, out_hbm, tab_v, idx_v, out_v, lsem, osem):
    wid = lax.axis_index("s") * NC + lax.axis_index("c")

    def pair(k, fprev):
        p = wid * PPW + k
        f = lax.div(p, D)
        d = lax.rem(p, D)
        tcp = pltpu.make_async_copy(tw_hbm.at[f, d], tab_v, lsem)
        icp = pltpu.make_async_copy(xt_hbm.at[f], idx_v, lsem)
        newf = f != fprev

        @pl.when(newf)
        def _():
            icp.start()

        tcp.start()
        tcp.wait()

        @pl.when(newf)
        def _():
            icp.wait()

        for oc in range(NOC):
            b = oc % 2
            ocp = pltpu.make_async_copy(
                out_v.at[b], out_hbm.at[p, pl.ds(oc * OC, OC)], osem.at[b]
            )

            if oc >= 2:
                ocp.wait()
            else:

                @pl.when(k > 0)
                def _():
                    ocp.wait()

            @plsc.parallel_loop(0, OC, L, unroll=16)
            def gather(j):
                o = pl.ds(j, L)
                out_v[b, o] = plsc.load_gather(
                    tab_v, [idx_v[pl.ds(oc * OC + j, L)]]
                )
            ocp.start()
        return f

    lax.fori_loop(0, PPW, pair, jnp.int32(-1))

    for b in range(2):
        pltpu.make_async_copy(
            out_v.at[b], out_hbm.at[0, pl.ds(b * OC, OC)], osem.at[b]
        ).wait()


def kernel(x, tables):
    xt = x.T.astype(jnp.int32)
    tw = jnp.transpose(tables, (0, 2, 1))
    out_t = _emb(xt, tw)
    return out_t.T


# speedup vs baseline: 1.0022x; 1.0022x over previous
import functools

import jax
import jax.numpy as jnp
from jax import lax
from jax.experimental import pallas as pl
from jax.experimental.pallas import tpu as pltpu
from jax.experimental.pallas import tpu_sc as plsc

BATCH = 16384
F = 26
V = 100000
D = 16

NC, NS, L = 2, 16, 16
NW = NC * NS
NP = F * D
PPW = NP // NW
OC = 4096
NOC = BATCH // OC

_mesh = plsc.VectorSubcoreMesh(
    core_axis_name="c", subcore_axis_name="s", num_cores=NC, num_subcores=NS
)


@functools.partial(
    pl.kernel,
    out_type=jax.ShapeDtypeStruct((NP, BATCH), jnp.float32),
    mesh=_mesh,
    scratch_types=[
        pltpu.VMEM((V,), jnp.float32),
        pltpu.VMEM((BATCH,), jnp.int32),
        pltpu.VMEM((2, OC), jnp.float32),
        pltpu.SemaphoreType.DMA,
        pltpu.SemaphoreType.DMA((2,)),
    ],
    compiler_params=pltpu.CompilerParams(
        use_tc_tiling_on_sc=True, needs_layout_passes=False
    ),
)
def _emb(xt_hbm, tw_hbm, out_hbm, tab_v, idx_v, out_v, lsem, osem):
    wid = lax.axis_index("s") * NC + lax.axis_index("c")

    def pair(k, fprev):
        p = wid * PPW + k
        f = lax.div(p, D)
        d = lax.rem(p, D)
        tcp = pltpu.make_async_copy(tw_hbm.at[f, d], tab_v, lsem)
        icp = pltpu.make_async_copy(xt_hbm.at[f], idx_v, lsem)
        newf = f != fprev

        @pl.when(newf)
        def _():
            icp.start()

        tcp.start()
        tcp.wait()

        @pl.when(newf)
        def _():
            icp.wait()

        for oc in range(NOC):
            b = oc % 2
            ocp = pltpu.make_async_copy(
                out_v.at[b], out_hbm.at[p, pl.ds(oc * OC, OC)], osem.at[b]
            )

            if oc >= 2:
                ocp.wait()
            else:

                @pl.when(k > 0)
                def _():
                    ocp.wait()

            @plsc.parallel_loop(0, OC, L, unroll=8)
            def gather(j):
                o = pl.ds(j, L)
                out_v[b, o] = plsc.load_gather(
                    tab_v, [idx_v[pl.ds(oc * OC + j, L)]]
                )
            ocp.start()
        return f

    lax.fori_loop(0, PPW, pair, jnp.int32(-1))

    for b in range(2):
        pltpu.make_async_copy(
            out_v.at[b], out_hbm.at[0, pl.ds(b * OC, OC)], osem.at[b]
        ).wait()


def kernel(x, tables):
    xt = x.T.astype(jnp.int32)
    tw = jnp.transpose(tables, (0, 2, 1))
    out_t = _emb(xt, tw)
    return out_t.T
